# baseline (device time: 49404 ns/iter reference)
import jax
import jax.numpy as jnp
from jax import lax
from jax.experimental import pallas as pl
from jax.experimental.pallas import tpu as pltpu

N_DEV = 4
B = 2
SQ = 512
SKV = 512
H_LOC = 8
DH = 64
BLK = 64
D_MODEL = 768
CHUNK = H_LOC * DH
HALF = CHUNK // 2

_QSCALE = 0.125 * 1.4426950408889634


def kernel(x, Wq, K_ext, V_ext, Wo):
    def body(x_ref, wq_ref, k_ref, v_ref, wo_ref, out_ref,
             full_ref, half_ref, send_sems, recv_sems):
        me = lax.axis_index("i")
        left = lax.rem(me - 1 + N_DEV, N_DEV)
        right = lax.rem(me + 1, N_DEV)
        opp = lax.rem(me + 2, N_DEV)

        barrier_sem = pltpu.get_barrier_semaphore()
        for nbr in (left, right):
            pl.semaphore_signal(
                barrier_sem, inc=1,
                device_id=(nbr,), device_id_type=pl.DeviceIdType.MESH,
            )
        pl.semaphore_wait(barrier_sem, 2)

        qi = lax.broadcasted_iota(jnp.int32, (SQ, SKV), 0)
        kj = lax.broadcasted_iota(jnp.int32, (SQ, SKV), 1)
        mask = (kj // BLK) <= (qi // BLK)

        wq_me = (wq_ref[:, pl.ds(me * CHUNK, CHUNK)] * _QSCALE
                 ).astype(jnp.bfloat16)
        for b in range(B):
            xb = x_ref[b].astype(jnp.bfloat16)
            qb = jnp.dot(xb, wq_me, preferred_element_type=jnp.float32)
            for h in range(H_LOC):
                q = qb[:, h * DH:(h + 1) * DH].astype(jnp.bfloat16)
                k = k_ref[b, :, h, :].astype(jnp.bfloat16)
                v = v_ref[b, :, h, :].astype(jnp.bfloat16)
                s = lax.dot_general(
                    q, k, (((1,), (1,)), ((), ())),
                    preferred_element_type=jnp.float32,
                )
                e = jnp.exp2(jnp.where(mask, s, -1e9))
                recip = 1.0 / jnp.sum(e, axis=-1, keepdims=True)
                ctx = jnp.dot(e.astype(jnp.bfloat16), v,
                              preferred_element_type=jnp.float32) * recip
                full_ref[0, b, :, h * DH:(h + 1) * DH] = ctx.astype(jnp.bfloat16)

        t1r = pltpu.make_async_remote_copy(
            src_ref=full_ref.at[0], dst_ref=full_ref.at[1],
            send_sem=send_sems.at[0], recv_sem=recv_sems.at[0],
            device_id=(right,), device_id_type=pl.DeviceIdType.MESH,
        )
        t1l = pltpu.make_async_remote_copy(
            src_ref=full_ref.at[0], dst_ref=full_ref.at[2],
            send_sem=send_sems.at[1], recv_sem=recv_sems.at[1],
            device_id=(left,), device_id_type=pl.DeviceIdType.MESH,
        )
        t1r.start()
        t1l.start()

        wo_me = wo_ref[pl.ds(me * CHUNK, CHUNK), :].astype(jnp.bfloat16)
        for b in range(B):
            out_ref[b] = jnp.dot(full_ref[0, b], wo_me,
                                 preferred_element_type=jnp.float32)

        t1r.wait_recv()
        t2r = pltpu.make_async_remote_copy(
            src_ref=full_ref.at[1, :, :, pl.ds(0, HALF)],
            dst_ref=half_ref.at[0],
            send_sem=send_sems.at[2], recv_sem=recv_sems.at[2],
            device_id=(right,), device_id_type=pl.DeviceIdType.MESH,
        )
        t2r.start()
        t1l.wait_recv()
        t2l = pltpu.make_async_remote_copy(
            src_ref=full_ref.at[2, :, :, pl.ds(HALF, HALF)],
            dst_ref=half_ref.at[1],
            send_sem=send_sems.at[3], recv_sem=recv_sems.at[3],
            device_id=(left,), device_id_type=pl.DeviceIdType.MESH,
        )
        t2l.start()

        wo_l = wo_ref[pl.ds(left * CHUNK, CHUNK), :].astype(jnp.bfloat16)
        wo_r = wo_ref[pl.ds(right * CHUNK, CHUNK), :].astype(jnp.bfloat16)
        for b in range(B):
            out_ref[b] = out_ref[b] + jnp.dot(
                full_ref[1, b], wo_l, preferred_element_type=jnp.float32
            ) + jnp.dot(
                full_ref[2, b], wo_r, preferred_element_type=jnp.float32
            )

        t2r.wait_recv()
        t2l.wait_recv()
        wo_oa = wo_ref[pl.ds(opp * CHUNK, HALF), :].astype(jnp.bfloat16)
        wo_ob = wo_ref[pl.ds(opp * CHUNK + HALF, HALF), :].astype(jnp.bfloat16)
        for b in range(B):
            out_ref[b] = out_ref[b] + jnp.dot(
                half_ref[0, b], wo_oa, preferred_element_type=jnp.float32
            ) + jnp.dot(
                half_ref[1, b], wo_ob, preferred_element_type=jnp.float32
            )

        t1r.wait_send()
        t1l.wait_send()
        t2r.wait_send()
        t2l.wait_send()

    return pl.pallas_call(
        body,
        out_shape=jax.ShapeDtypeStruct((B, SQ, D_MODEL), jnp.float32),
        in_specs=[pl.BlockSpec(memory_space=pltpu.VMEM)] * 5,
        out_specs=pl.BlockSpec(memory_space=pltpu.VMEM),
        scratch_shapes=[
            pltpu.VMEM((3, B, SQ, CHUNK), jnp.bfloat16),
            pltpu.VMEM((2, B, SQ, HALF), jnp.bfloat16),
            pltpu.SemaphoreType.DMA((4,)),
            pltpu.SemaphoreType.DMA((4,)),
        ],
        compiler_params=pltpu.CompilerParams(collective_id=0),
    )(x, Wq, K_ext, V_ext, Wo)


# device time: 42622 ns/iter; 1.1591x vs baseline; 1.1591x over previous
import jax
import jax.numpy as jnp
from jax import lax
from jax.experimental import pallas as pl
from jax.experimental.pallas import tpu as pltpu

N_DEV = 4
B = 2
SQ = 512
SKV = 512
H_LOC = 8
DH = 64
BLK = 64
D_MODEL = 768
CHUNK = H_LOC * DH
HALF = CHUNK // 2
TOP = SQ // 2

_QSCALE = 0.125 * 1.4426950408889634


def kernel(x, Wq, K_ext, V_ext, Wo):
    def body(x_ref, wq_ref, k_ref, v_ref, wo_ref, out_ref,
             full_ref, half_ref, send_sems, recv_sems):
        me = lax.axis_index("i")
        left = lax.rem(me - 1 + N_DEV, N_DEV)
        right = lax.rem(me + 1, N_DEV)
        opp = lax.rem(me + 2, N_DEV)

        barrier_sem = pltpu.get_barrier_semaphore()
        for nbr in (left, right):
            pl.semaphore_signal(
                barrier_sem, inc=1,
                device_id=(nbr,), device_id_type=pl.DeviceIdType.MESH,
            )
        pl.semaphore_wait(barrier_sem, 2)

        qi_t = lax.broadcasted_iota(jnp.int32, (TOP, TOP), 0)
        kj_t = lax.broadcasted_iota(jnp.int32, (TOP, TOP), 1)
        mask_top = (kj_t // BLK) <= (qi_t // BLK)
        qi_b = TOP + lax.broadcasted_iota(jnp.int32, (SQ - TOP, SKV), 0)
        kj_b = lax.broadcasted_iota(jnp.int32, (SQ - TOP, SKV), 1)
        mask_bot = (kj_b // BLK) <= (qi_b // BLK)

        wq_me = (wq_ref[:, pl.ds(me * CHUNK, CHUNK)] * _QSCALE
                 ).astype(jnp.bfloat16)
        qs = [
            jnp.dot(x_ref[b].astype(jnp.bfloat16), wq_me,
                    preferred_element_type=jnp.float32)
            for b in range(B)
        ]

        def attend(b, h):
            q = qs[b][:, h * DH:(h + 1) * DH].astype(jnp.bfloat16)
            k = k_ref[b, :, h, :].astype(jnp.bfloat16)
            v = v_ref[b, :, h, :].astype(jnp.bfloat16)
            s_t = lax.dot_general(
                q[0:TOP], k[0:TOP], (((1,), (1,)), ((), ())),
                preferred_element_type=jnp.float32,
            )
            e_t = jnp.exp2(jnp.where(mask_top, s_t, -1e9))
            r_t = 1.0 / jnp.sum(e_t, axis=-1, keepdims=True)
            ctx_t = jnp.dot(e_t.astype(jnp.bfloat16), v[0:TOP],
                            preferred_element_type=jnp.float32) * r_t
            s_b = lax.dot_general(
                q[TOP:], k, (((1,), (1,)), ((), ())),
                preferred_element_type=jnp.float32,
            )
            e_b = jnp.exp2(jnp.where(mask_bot, s_b, -1e9))
            r_b = 1.0 / jnp.sum(e_b, axis=-1, keepdims=True)
            ctx_b = jnp.dot(e_b.astype(jnp.bfloat16), v,
                            preferred_element_type=jnp.float32) * r_b
            col = h * DH
            full_ref[0, b, 0:TOP, col:col + DH] = ctx_t.astype(jnp.bfloat16)
            full_ref[0, b, TOP:, col:col + DH] = ctx_b.astype(jnp.bfloat16)

        def mk(src, dst, sem, dev):
            return pltpu.make_async_remote_copy(
                src_ref=src, dst_ref=dst,
                send_sem=send_sems.at[sem], recv_sem=recv_sems.at[sem],
                device_id=(dev,), device_id_type=pl.DeviceIdType.MESH,
            )

        for b in range(B):
            for h in range(H_LOC // 2):
                attend(b, h)
        sAr = mk(full_ref.at[0, :, :, pl.ds(0, HALF)],
                 full_ref.at[1, :, :, pl.ds(0, HALF)], 0, right)
        sAl = mk(full_ref.at[0, :, :, pl.ds(0, HALF)],
                 full_ref.at[2, :, :, pl.ds(0, HALF)], 1, left)
        sAr.start()
        sAl.start()

        for b in range(B):
            for h in range(H_LOC // 2, H_LOC):
                attend(b, h)
        sBr = mk(full_ref.at[0, :, :, pl.ds(HALF, HALF)],
                 full_ref.at[1, :, :, pl.ds(HALF, HALF)], 2, right)
        sBl = mk(full_ref.at[0, :, :, pl.ds(HALF, HALF)],
                 full_ref.at[2, :, :, pl.ds(HALF, HALF)], 3, left)
        sBr.start()
        sBl.start()

        wo_me = wo_ref[pl.ds(me * CHUNK, CHUNK), :].astype(jnp.bfloat16)
        for b in range(B):
            out_ref[b] = jnp.dot(full_ref[0, b], wo_me,
                                 preferred_element_type=jnp.float32)

        sAr.wait_recv()
        fAr = mk(full_ref.at[1, :, :, pl.ds(0, HALF)], half_ref.at[0],
                 4, right)
        fAr.start()
        sBl.wait_recv()
        fBl = mk(full_ref.at[2, :, :, pl.ds(HALF, HALF)], half_ref.at[1],
                 5, left)
        fBl.start()

        sAl.wait_recv()
        sBr.wait_recv()
        wo_l = wo_ref[pl.ds(left * CHUNK, CHUNK), :].astype(jnp.bfloat16)
        wo_r = wo_ref[pl.ds(right * CHUNK, CHUNK), :].astype(jnp.bfloat16)
        for b in range(B):
            out_ref[b] = out_ref[b] + jnp.dot(
                full_ref[1, b], wo_l, preferred_element_type=jnp.float32
            ) + jnp.dot(
                full_ref[2, b], wo_r, preferred_element_type=jnp.float32
            )

        fAr.wait_recv()
        fBl.wait_recv()
        wo_oa = wo_ref[pl.ds(opp * CHUNK, HALF), :].astype(jnp.bfloat16)
        wo_ob = wo_ref[pl.ds(opp * CHUNK + HALF, HALF), :].astype(jnp.bfloat16)
        for b in range(B):
            out_ref[b] = out_ref[b] + jnp.dot(
                half_ref[0, b], wo_oa, preferred_element_type=jnp.float32
            ) + jnp.dot(
                half_ref[1, b], wo_ob, preferred_element_type=jnp.float32
            )

        for t in (sAr, sAl, sBr, sBl, fAr, fBl):
            t.wait_send()

    return pl.pallas_call(
        body,
        out_shape=jax.ShapeDtypeStruct((B, SQ, D_MODEL), jnp.float32),
        in_specs=[pl.BlockSpec(memory_space=pltpu.VMEM)] * 5,
        out_specs=pl.BlockSpec(memory_space=pltpu.VMEM),
        scratch_shapes=[
            pltpu.VMEM((3, B, SQ, CHUNK), jnp.bfloat16),
            pltpu.VMEM((2, B, SQ, HALF), jnp.bfloat16),
            pltpu.SemaphoreType.DMA((6,)),
            pltpu.SemaphoreType.DMA((6,)),
        ],
        compiler_params=pltpu.CompilerParams(collective_id=0),
    )(x, Wq, K_ext, V_ext, Wo)


# device time: 39781 ns/iter; 1.2419x vs baseline; 1.0714x over previous
import jax
import jax.numpy as jnp
from jax import lax
from jax.experimental import pallas as pl
from jax.experimental.pallas import tpu as pltpu

N_DEV = 4
B = 2
SQ = 512
SKV = 512
H_LOC = 8
DH = 64
BLK = 64
D_MODEL = 768
CHUNK = H_LOC * DH
QTR = CHUNK // 4
TOP = SQ // 2

_QSCALE = 0.125 * 1.4426950408889634


def kernel(x, Wq, K_ext, V_ext, Wo):
    def body(x_ref, wq_ref, k_ref, v_ref, wo_ref, out_ref,
             full_ref, send_sems, recv_sems):
        me = lax.axis_index("i")
        left = lax.rem(me - 1 + N_DEV, N_DEV)
        right = lax.rem(me + 1, N_DEV)
        opp = lax.rem(me + 2, N_DEV)

        barrier_sem = pltpu.get_barrier_semaphore()
        for nbr in (left, right):
            pl.semaphore_signal(
                barrier_sem, inc=1,
                device_id=(nbr,), device_id_type=pl.DeviceIdType.MESH,
            )
        pl.semaphore_wait(barrier_sem, 2)

        qi_t = lax.broadcasted_iota(jnp.int32, (TOP, TOP), 0)
        kj_t = lax.broadcasted_iota(jnp.int32, (TOP, TOP), 1)
        mask_top = (kj_t // BLK) <= (qi_t // BLK)
        qi_b = TOP + lax.broadcasted_iota(jnp.int32, (SQ - TOP, SKV), 0)
        kj_b = lax.broadcasted_iota(jnp.int32, (SQ - TOP, SKV), 1)
        mask_bot = (kj_b // BLK) <= (qi_b // BLK)

        wq_me = (wq_ref[:, pl.ds(me * CHUNK, CHUNK)] * _QSCALE
                 ).astype(jnp.bfloat16)
        qs = [
            jnp.dot(x_ref[b].astype(jnp.bfloat16), wq_me,
                    preferred_element_type=jnp.float32)
            for b in range(B)
        ]

        def attend(b, h):
            q = qs[b][:, h * DH:(h + 1) * DH].astype(jnp.bfloat16)
            k = k_ref[b, :, h, :].astype(jnp.bfloat16)
            v = v_ref[b, :, h, :].astype(jnp.bfloat16)
            s_t = lax.dot_general(
                q[0:TOP], k[0:TOP], (((1,), (1,)), ((), ())),
                preferred_element_type=jnp.float32,
            )
            e_t = jnp.exp2(jnp.where(mask_top, s_t, -1e9))
            r_t = 1.0 / jnp.sum(e_t, axis=-1, keepdims=True)
            ctx_t = jnp.dot(e_t.astype(jnp.bfloat16), v[0:TOP],
                            preferred_element_type=jnp.float32) * r_t
            s_b = lax.dot_general(
                q[TOP:], k, (((1,), (1,)), ((), ())),
                preferred_element_type=jnp.float32,
            )
            e_b = jnp.exp2(jnp.where(mask_bot, s_b, -1e9))
            r_b = 1.0 / jnp.sum(e_b, axis=-1, keepdims=True)
            ctx_b = jnp.dot(e_b.astype(jnp.bfloat16), v,
                            preferred_element_type=jnp.float32) * r_b
            col = h * DH
            full_ref[0, b, 0:TOP, col:col + DH] = ctx_t.astype(jnp.bfloat16)
            full_ref[0, b, TOP:, col:col + DH] = ctx_b.astype(jnp.bfloat16)

        def mk(slot_src, slot_dst, q, sem, dev):
            return pltpu.make_async_remote_copy(
                src_ref=full_ref.at[slot_src, :, :, pl.ds(q * QTR, QTR)],
                dst_ref=full_ref.at[slot_dst, :, :, pl.ds(q * QTR, QTR)],
                send_sem=send_sems.at[sem], recv_sem=recv_sems.at[sem],
                device_id=(dev,), device_id_type=pl.DeviceIdType.MESH,
            )

        dQr = [mk(0, 1, q, q, right) for q in range(4)]
        dQl = [mk(0, 2, q, 4 + q, left) for q in range(4)]
        fQ1r = mk(1, 3, 0, 8, right)
        fQ3r = mk(1, 3, 2, 9, right)
        fQ2l = mk(2, 3, 1, 10, left)
        fQ4l = mk(2, 3, 3, 11, left)

        for b in range(B):
            attend(b, 0)
            attend(b, 1)
        dQr[0].start()
        dQl[0].start()
        for b in range(B):
            attend(b, 2)
            attend(b, 3)
        dQr[1].start()
        dQl[1].start()
        for b in range(B):
            attend(b, 4)
            attend(b, 5)
        dQr[2].start()
        dQl[2].start()
        dQr[0].wait_recv()
        fQ1r.start()
        dQl[1].wait_recv()
        fQ2l.start()
        for b in range(B):
            attend(b, 6)
            attend(b, 7)
        dQr[3].start()
        dQl[3].start()
        dQr[2].wait_recv()
        fQ3r.start()

        wo_me = wo_ref[pl.ds(me * CHUNK, CHUNK), :].astype(jnp.bfloat16)
        for b in range(B):
            out_ref[b] = jnp.dot(full_ref[0, b], wo_me,
                                 preferred_element_type=jnp.float32)

        dQl[3].wait_recv()
        fQ4l.start()

        dQr[1].wait_recv()
        dQr[3].wait_recv()
        dQl[0].wait_recv()
        dQl[2].wait_recv()
        wo_l = wo_ref[pl.ds(left * CHUNK, CHUNK), :].astype(jnp.bfloat16)
        wo_r = wo_ref[pl.ds(right * CHUNK, CHUNK), :].astype(jnp.bfloat16)
        for b in range(B):
            out_ref[b] = out_ref[b] + jnp.dot(
                full_ref[1, b], wo_l, preferred_element_type=jnp.float32
            ) + jnp.dot(
                full_ref[2, b], wo_r, preferred_element_type=jnp.float32
            )

        fQ1r.wait_recv()
        fQ2l.wait_recv()
        fQ3r.wait_recv()
        fQ4l.wait_recv()
        wo_o = wo_ref[pl.ds(opp * CHUNK, CHUNK), :].astype(jnp.bfloat16)
        for b in range(B):
            out_ref[b] = out_ref[b] + jnp.dot(
                full_ref[3, b], wo_o, preferred_element_type=jnp.float32
            )

        for t in dQr + dQl + [fQ1r, fQ3r, fQ2l, fQ4l]:
            t.wait_send()

    return pl.pallas_call(
        body,
        out_shape=jax.ShapeDtypeStruct((B, SQ, D_MODEL), jnp.float32),
        in_specs=[pl.BlockSpec(memory_space=pltpu.VMEM)] * 5,
        out_specs=pl.BlockSpec(memory_space=pltpu.VMEM),
        scratch_shapes=[
            pltpu.VMEM((4, B, SQ, CHUNK), jnp.bfloat16),
            pltpu.SemaphoreType.DMA((12,)),
            pltpu.SemaphoreType.DMA((12,)),
        ],
        compiler_params=pltpu.CompilerParams(collective_id=0),
    )(x, Wq, K_ext, V_ext, Wo)
